# baseline (speedup 1.0000x reference)
import functools

import jax
import jax.numpy as jnp
from jax import lax
from jax.experimental import pallas as pl
from jax.experimental.pallas import tpu as pltpu
from jax.experimental.pallas import tpu_sc as plsc

NUM_RETRIEVAL = 4
KINEMATIC_COEF = 0.1
EPS = 1e-8
KEY_BLOCK = 2048


def _make_score_topk(B, D, K, BK, QB):
    nblk = pl.cdiv(K, BK)
    nq = B // QB

    def body(q_ref, k_ref, ml_ref, len_ref, vals_ref, idx_ref, qn_ref):
        i = pl.program_id(1)

        @pl.when(i == 0)
        def _init():
            q = q_ref[...]
            qn_ref[...] = q / (jnp.sqrt(jnp.sum(q * q, axis=1, keepdims=True)) + EPS)
            vals_ref[...] = jnp.full(vals_ref.shape, -jnp.inf, dtype=jnp.float32)
            idx_ref[...] = jnp.zeros(idx_ref.shape, dtype=jnp.int32)

        k = k_ref[...]
        kn = k / (jnp.sqrt(jnp.sum(k * k, axis=1, keepdims=True)) + EPS)
        sem = lax.dot_general(qn_ref[...], kn, (((1,), (1,)), ((), ())),
                              preferred_element_type=jnp.float32)

        mlf = jnp.maximum(ml_ref[...].astype(jnp.float32), 1.0)
        lf = jnp.maximum(len_ref[...].astype(jnp.float32), 1.0)
        rel = jnp.abs(mlf - lf) / jnp.maximum(mlf, lf)
        score = sem * jnp.exp(rel * (-KINEMATIC_COEF))

        lcol = lax.broadcasted_iota(jnp.int32, (QB, BK), 1)
        score = lax.cond(
            i == nblk - 1,
            lambda sc: jnp.where(lcol < K - i * BK, sc, -jnp.inf),
            lambda sc: sc,
            score)

        INT_BIG = jnp.int32(2**31 - 1)
        bv, bi = [], []
        s = score
        for _ in range(NUM_RETRIEVAL):
            m = jnp.max(s, axis=1, keepdims=True)
            am = jnp.min(jnp.where(s == m, lcol, INT_BIG), axis=1, keepdims=True)
            bv.append(m)
            bi.append(am)
            s = jnp.where(lcol == am, -jnp.inf, s)
        blk_v = jnp.concatenate(bv, axis=1)
        blk_i = jnp.concatenate(bi, axis=1) + i * BK

        cand_v = jnp.concatenate([vals_ref[...], blk_v], axis=1)
        cand_i = jnp.concatenate([idx_ref[...], blk_i], axis=1)
        pos = lax.broadcasted_iota(jnp.int32, (QB, 2 * NUM_RETRIEVAL), 1)
        nv, ni = [], []
        v = cand_v
        for _ in range(NUM_RETRIEVAL):
            m = jnp.max(v, axis=1, keepdims=True)
            p = jnp.min(jnp.where(v == m, pos, INT_BIG), axis=1, keepdims=True)
            sel = pos == p
            nv.append(m)
            ni.append(jnp.sum(jnp.where(sel, cand_i, 0), axis=1, keepdims=True))
            v = jnp.where(sel, -jnp.inf, v)
        vals_ref[...] = jnp.concatenate(nv, axis=1)
        idx_ref[...] = jnp.concatenate(ni, axis=1)

    return pl.pallas_call(
        body,
        grid=(nq, nblk),
        in_specs=[
            pl.BlockSpec((QB, D), lambda iq, i: (iq, 0)),
            pl.BlockSpec((BK, D), lambda iq, i: (i, 0)),
            pl.BlockSpec((1, BK), lambda iq, i: (0, i)),
            pl.BlockSpec((QB, 1), lambda iq, i: (iq, 0)),
        ],
        out_specs=[
            pl.BlockSpec((QB, NUM_RETRIEVAL), lambda iq, i: (iq, 0)),
            pl.BlockSpec((QB, NUM_RETRIEVAL), lambda iq, i: (iq, 0)),
        ],
        out_shape=[
            jax.ShapeDtypeStruct((B, NUM_RETRIEVAL), jnp.float32),
            jax.ShapeDtypeStruct((B, NUM_RETRIEVAL), jnp.int32),
        ],
        scratch_shapes=[pltpu.VMEM((QB, D), jnp.float32)],
        compiler_params=pltpu.CompilerParams(
            dimension_semantics=("parallel", "arbitrary")),
    )


def _make_project(N, D):
    def body(g_ref, w_ref, b_ref, o_ref):
        o_ref[...] = lax.dot_general(
            g_ref[...], w_ref[...], (((1,), (0,)), ((), ())),
            preferred_element_type=jnp.float32) + b_ref[...]

    return pl.pallas_call(
        body,
        out_shape=jax.ShapeDtypeStruct((N, D), jnp.float32),
    )


def _sc_gather(table, idx_flat):
    N = idx_flat.shape[0]
    D = table.shape[1]
    info = plsc.get_sparse_core_info()
    NC, NS = info.num_cores, info.num_subcores
    NW = NC * NS
    b_per_w = N // NW
    mesh = plsc.VectorSubcoreMesh(core_axis_name="c", subcore_axis_name="s")

    @functools.partial(
        pl.kernel,
        mesh=mesh,
        out_type=jax.ShapeDtypeStruct((N, D), jnp.float32),
        scratch_types=[
            pltpu.VMEM((b_per_w,), jnp.int32),
            pltpu.VMEM((b_per_w, D), jnp.float32),
            pltpu.SemaphoreType.DMA,
        ],
    )
    def gather_k(table_hbm, idx_hbm, out_hbm, idx_v, rows_v, sem):
        wid = lax.axis_index("s") * NC + lax.axis_index("c")
        base = wid * b_per_w
        pltpu.sync_copy(idx_hbm.at[pl.ds(base, b_per_w)], idx_v)
        pltpu.async_copy(table_hbm.at[idx_v], rows_v, sem).wait()
        pltpu.sync_copy(rows_v, out_hbm.at[pl.ds(base, b_per_w)])

    return gather_k(table, idx_flat)


def kernel(query_features, text_features, m_lengths, lengths, W_proj, b_proj):
    B, D = query_features.shape
    K = text_features.shape[0]
    score_topk = _make_score_topk(B, D, K, KEY_BLOCK, B // 2)
    top_scores, top_idx = score_topk(
        query_features, text_features,
        m_lengths.reshape(1, K), lengths.reshape(B, 1))
    idx_flat = top_idx.reshape(-1)
    gathered = _sc_gather(text_features, idx_flat)
    re_flat = _make_project(B * NUM_RETRIEVAL, D)(
        gathered, W_proj, b_proj.reshape(1, D))
    return top_scores, top_idx, re_flat.reshape(B, NUM_RETRIEVAL, D)


# speedup vs baseline: 1.5128x; 1.5128x over previous
import functools

import jax
import jax.numpy as jnp
from jax import lax
from jax.experimental import pallas as pl
from jax.experimental.pallas import tpu as pltpu
from jax.experimental.pallas import tpu_sc as plsc

NUM_RETRIEVAL = 4
KINEMATIC_COEF = 0.1
EPS = 1e-8
KEY_BLOCK = 2048


def _make_score_topk(B, D, K, BK):
    nblk = pl.cdiv(K, BK)

    BIGF = 3.0e8

    def body(q_ref, k_ref, ml_ref, len_ref, ov_ref, oi_ref, qn_ref):
        i = pl.program_id(0)

        @pl.when(i == 0)
        def _init():
            q = q_ref[...]
            qn_ref[...] = q / (jnp.sqrt(jnp.sum(q * q, axis=1, keepdims=True)) + EPS)

        k = k_ref[...]
        kn = k / (jnp.sqrt(jnp.sum(k * k, axis=1, keepdims=True)) + EPS)
        sem = lax.dot_general(qn_ref[...], kn, (((1,), (1,)), ((), ())),
                              preferred_element_type=jnp.float32)

        mlf = jnp.maximum(ml_ref[...].astype(jnp.float32), 1.0)
        lf = jnp.maximum(len_ref[...].astype(jnp.float32), 1.0)
        rel = jnp.abs(mlf - lf) / jnp.maximum(mlf, lf)
        score = sem * jnp.exp(rel * (-KINEMATIC_COEF))

        lcol = lax.broadcasted_iota(jnp.int32, (B, BK), 1).astype(jnp.float32)
        score = lax.cond(
            i == nblk - 1,
            lambda sc: jnp.where(lcol < jnp.float32(K - i * BK), sc, -jnp.inf),
            lambda sc: sc,
            score)

        bv, bi = [], []
        s = score
        for _ in range(NUM_RETRIEVAL):
            m = jnp.max(s, axis=1, keepdims=True)
            am = jnp.min(jnp.where(s == m, lcol, BIGF), axis=1, keepdims=True)
            bv.append(m)
            bi.append(am)
            s = jnp.where(lcol == am, -jnp.inf, s)
        blk_v = jnp.concatenate(bv, axis=1)
        blk_i = jnp.concatenate(bi, axis=1) + jnp.float32(i * BK)

        ov_ref[...] = blk_v.reshape(1, B, NUM_RETRIEVAL)
        oi_ref[...] = blk_i.reshape(1, B, NUM_RETRIEVAL)

    return pl.pallas_call(
        body,
        grid=(nblk,),
        in_specs=[
            pl.BlockSpec((B, D), lambda i: (0, 0)),
            pl.BlockSpec((BK, D), lambda i: (i, 0)),
            pl.BlockSpec((1, BK), lambda i: (0, i)),
            pl.BlockSpec((B, 1), lambda i: (0, 0)),
        ],
        out_specs=[
            pl.BlockSpec((1, B, NUM_RETRIEVAL), lambda i: (i, 0, 0)),
            pl.BlockSpec((1, B, NUM_RETRIEVAL), lambda i: (i, 0, 0)),
        ],
        out_shape=[
            jax.ShapeDtypeStruct((nblk, B, NUM_RETRIEVAL), jnp.float32),
            jax.ShapeDtypeStruct((nblk, B, NUM_RETRIEVAL), jnp.float32),
        ],
        scratch_shapes=[pltpu.VMEM((B, D), jnp.float32)],
    )


def _make_merge(B, NC):
    BIGF = 3.0e8

    def body(v_ref, i_ref, vals_ref, idx_ref):
        V = v_ref[...]
        I = i_ref[...]
        fv, fi = [], []
        for _ in range(NUM_RETRIEVAL):
            m = jnp.max(V, axis=1, keepdims=True)
            p = jnp.min(jnp.where(V == m, I, BIGF), axis=1, keepdims=True)
            fv.append(m)
            fi.append(p)
            V = jnp.where(I == p, -jnp.inf, V)
        vals_ref[...] = jnp.concatenate(fv, axis=1)
        idx_ref[...] = jnp.concatenate(fi, axis=1).astype(jnp.int32)

    return pl.pallas_call(
        body,
        out_shape=[
            jax.ShapeDtypeStruct((B, NUM_RETRIEVAL), jnp.float32),
            jax.ShapeDtypeStruct((B, NUM_RETRIEVAL), jnp.int32),
        ],
    )


def _make_project(N, D):
    def body(g_ref, w_ref, b_ref, o_ref):
        o_ref[...] = lax.dot_general(
            g_ref[...], w_ref[...], (((1,), (0,)), ((), ())),
            preferred_element_type=jnp.float32) + b_ref[...]

    return pl.pallas_call(
        body,
        out_shape=jax.ShapeDtypeStruct((N, D), jnp.float32),
    )


def _sc_gather(table, idx_flat):
    N = idx_flat.shape[0]
    D = table.shape[1]
    info = plsc.get_sparse_core_info()
    NC, NS = info.num_cores, info.num_subcores
    NW = NC * NS
    b_per_w = N // NW
    mesh = plsc.VectorSubcoreMesh(core_axis_name="c", subcore_axis_name="s")

    @functools.partial(
        pl.kernel,
        mesh=mesh,
        out_type=jax.ShapeDtypeStruct((N, D), jnp.float32),
        scratch_types=[
            pltpu.VMEM((b_per_w,), jnp.int32),
            pltpu.VMEM((b_per_w, D), jnp.float32),
            pltpu.SemaphoreType.DMA,
        ],
    )
    def gather_k(table_hbm, idx_hbm, out_hbm, idx_v, rows_v, sem):
        wid = lax.axis_index("s") * NC + lax.axis_index("c")
        base = wid * b_per_w
        pltpu.sync_copy(idx_hbm.at[pl.ds(base, b_per_w)], idx_v)
        pltpu.async_copy(table_hbm.at[idx_v], rows_v, sem).wait()
        pltpu.sync_copy(rows_v, out_hbm.at[pl.ds(base, b_per_w)])

    return gather_k(table, idx_flat)


def kernel(query_features, text_features, m_lengths, lengths, W_proj, b_proj):
    B, D = query_features.shape
    K = text_features.shape[0]
    score_topk = _make_score_topk(B, D, K, KEY_BLOCK)
    cand_v, cand_i = score_topk(
        query_features, text_features,
        m_lengths.reshape(1, K), lengths.reshape(B, 1))
    nblk = cand_v.shape[0]
    nc = nblk * NUM_RETRIEVAL
    nc_pad = ((nc + 127) // 128) * 128
    cv = jnp.transpose(cand_v, (1, 0, 2)).reshape(B, nc)
    ci = jnp.transpose(cand_i, (1, 0, 2)).reshape(B, nc)
    cv = jnp.pad(cv, ((0, 0), (0, nc_pad - nc)), constant_values=-jnp.inf)
    ci = jnp.pad(ci, ((0, 0), (0, nc_pad - nc)), constant_values=3.0e8)
    top_scores, top_idx = _make_merge(B, nc_pad)(cv, ci)
    idx_flat = top_idx.reshape(-1)
    gathered = _sc_gather(text_features, idx_flat)
    re_flat = _make_project(B * NUM_RETRIEVAL, D)(
        gathered, W_proj, b_proj.reshape(1, D))
    return top_scores, top_idx, re_flat.reshape(B, NUM_RETRIEVAL, D)


# speedup vs baseline: 1.5148x; 1.0013x over previous
import functools

import jax
import jax.numpy as jnp
from jax import lax
from jax.experimental import pallas as pl
from jax.experimental.pallas import tpu as pltpu
from jax.experimental.pallas import tpu_sc as plsc

NUM_RETRIEVAL = 4
KINEMATIC_COEF = 0.1
EPS = 1e-8
KEY_BLOCK = 2048


def _make_score_topk(B, D, K, BK):
    nblk = pl.cdiv(K, BK)

    BIGF = 3.0e8

    def body(q_ref, k_ref, ml_ref, len_ref, ov_ref, oi_ref, qn_ref):
        i = pl.program_id(0)

        @pl.when(i == 0)
        def _init():
            q = q_ref[...]
            qn_ref[...] = q / (jnp.sqrt(jnp.sum(q * q, axis=1, keepdims=True)) + EPS)

        k = k_ref[...]
        kn = k / (jnp.sqrt(jnp.sum(k * k, axis=1, keepdims=True)) + EPS)
        sem = lax.dot_general(qn_ref[...], kn, (((1,), (1,)), ((), ())),
                              preferred_element_type=jnp.float32)

        mlf = jnp.maximum(ml_ref[...].astype(jnp.float32), 1.0)
        lf = jnp.maximum(len_ref[...].astype(jnp.float32), 1.0)
        rel = jnp.abs(mlf - lf) / jnp.maximum(mlf, lf)
        score = sem * jnp.exp(rel * (-KINEMATIC_COEF))

        lcol = lax.broadcasted_iota(jnp.int32, (B, BK), 1).astype(jnp.float32)
        score = lax.cond(
            i == nblk - 1,
            lambda sc: jnp.where(lcol < jnp.float32(K - i * BK), sc, -jnp.inf),
            lambda sc: sc,
            score)

        bv, bi = [], []
        s = score
        for _ in range(NUM_RETRIEVAL):
            m = jnp.max(s, axis=1, keepdims=True)
            am = jnp.min(jnp.where(s == m, lcol, BIGF), axis=1, keepdims=True)
            bv.append(m)
            bi.append(am)
            s = jnp.where(lcol == am, -jnp.inf, s)
        blk_v = jnp.concatenate(bv, axis=1)
        blk_i = jnp.concatenate(bi, axis=1) + jnp.float32(i * BK)

        ov_ref[...] = blk_v.reshape(1, B, NUM_RETRIEVAL)
        oi_ref[...] = blk_i.reshape(1, B, NUM_RETRIEVAL)

    return pl.pallas_call(
        body,
        grid=(nblk,),
        in_specs=[
            pl.BlockSpec((B, D), lambda i: (0, 0)),
            pl.BlockSpec((BK, D), lambda i: (i, 0)),
            pl.BlockSpec((1, BK), lambda i: (0, i)),
            pl.BlockSpec((B, 1), lambda i: (0, 0)),
        ],
        out_specs=[
            pl.BlockSpec((1, B, NUM_RETRIEVAL), lambda i: (i, 0, 0)),
            pl.BlockSpec((1, B, NUM_RETRIEVAL), lambda i: (i, 0, 0)),
        ],
        out_shape=[
            jax.ShapeDtypeStruct((nblk, B, NUM_RETRIEVAL), jnp.float32),
            jax.ShapeDtypeStruct((nblk, B, NUM_RETRIEVAL), jnp.float32),
        ],
        scratch_shapes=[pltpu.VMEM((B, D), jnp.float32)],
    )


def _make_merge(B, NC):
    BIGF = 3.0e8

    def body(v_ref, i_ref, vals_ref, idx_ref):
        V = v_ref[...]
        I = i_ref[...]
        fv, fi = [], []
        for _ in range(NUM_RETRIEVAL):
            m = jnp.max(V, axis=1, keepdims=True)
            p = jnp.min(jnp.where(V == m, I, BIGF), axis=1, keepdims=True)
            fv.append(m)
            fi.append(p)
            V = jnp.where(I == p, -jnp.inf, V)
        vals_ref[...] = jnp.concatenate(fv, axis=1)
        idx_ref[...] = jnp.concatenate(fi, axis=1).astype(jnp.int32)

    return pl.pallas_call(
        body,
        out_shape=[
            jax.ShapeDtypeStruct((B, NUM_RETRIEVAL), jnp.float32),
            jax.ShapeDtypeStruct((B, NUM_RETRIEVAL), jnp.int32),
        ],
    )


def _make_project(N, D):
    def body(g_ref, w_ref, b_ref, o_ref):
        o_ref[...] = lax.dot_general(
            g_ref[...], w_ref[...], (((1,), (0,)), ((), ())),
            preferred_element_type=jnp.float32) + b_ref[...]

    return pl.pallas_call(
        body,
        out_shape=jax.ShapeDtypeStruct((N, D), jnp.float32),
    )


def _sc_gather(table, idx_flat):
    N = idx_flat.shape[0]
    D = table.shape[1]
    info = plsc.get_sparse_core_info()
    NC, NS = info.num_cores, info.num_subcores
    NW = NC * NS
    b_per_w = N // NW
    mesh = plsc.VectorSubcoreMesh(core_axis_name="c", subcore_axis_name="s")

    @functools.partial(
        pl.kernel,
        mesh=mesh,
        out_type=jax.ShapeDtypeStruct((N, D), jnp.float32),
        scratch_types=[
            pltpu.VMEM((b_per_w,), jnp.int32),
            pltpu.VMEM((b_per_w, D), jnp.float32),
            pltpu.SemaphoreType.DMA,
        ],
    )
    def gather_k(table_hbm, idx_hbm, out_hbm, idx_v, rows_v, sem):
        wid = lax.axis_index("s") * NC + lax.axis_index("c")
        base = wid * b_per_w
        pltpu.sync_copy(idx_hbm.at[pl.ds(base, b_per_w)], idx_v)
        pltpu.async_copy(table_hbm.at[idx_v], rows_v, sem).wait()
        pltpu.sync_copy(rows_v, out_hbm.at[pl.ds(base, b_per_w)])

    return gather_k(table, idx_flat)


def kernel(query_features, text_features, m_lengths, lengths, W_proj, b_proj):
    B, D = query_features.shape
    K = text_features.shape[0]
    score_topk = _make_score_topk(B, D, K, KEY_BLOCK)
    cand_v, cand_i = score_topk(
        query_features, text_features,
        m_lengths.reshape(1, K), lengths.reshape(B, 1))
    nblk = cand_v.shape[0]
    nc = nblk * NUM_RETRIEVAL
    cv = jnp.transpose(cand_v, (1, 0, 2)).reshape(B, nc)
    ci = jnp.transpose(cand_i, (1, 0, 2)).reshape(B, nc)
    top_scores, top_idx = _make_merge(B, nc)(cv, ci)
    idx_flat = top_idx.reshape(-1)
    gathered = _sc_gather(text_features, idx_flat)
    re_flat = _make_project(B * NUM_RETRIEVAL, D)(
        gathered, W_proj, b_proj.reshape(1, D))
    return top_scores, top_idx, re_flat.reshape(B, NUM_RETRIEVAL, D)
